# baseline (speedup 1.0000x reference)
import functools
import math

import jax
import jax.numpy as jnp
from jax import lax
from jax.experimental import pallas as pl
from jax.experimental.pallas import tpu as pltpu
from jax.experimental.pallas import tpu_sc as plsc

D_MODEL = 512
SCALE = float(math.sqrt(D_MODEL))

NUM_CORES = 2
NUM_SUBCORES = 16
NW = NUM_CORES * NUM_SUBCORES

B_TOTAL = 1024 * 200
BPW = B_TOTAL // NW
C = 64
NCHUNK = BPW // C
NBUF = 3
PF = 2
T = NCHUNK // NBUF

_mesh = plsc.VectorSubcoreMesh(core_axis_name="c", subcore_axis_name="s")


@functools.partial(
    pl.kernel,
    mesh=_mesh,
    out_type=jax.ShapeDtypeStruct((B_TOTAL, D_MODEL), jnp.float32),
    scratch_types=(
        [pltpu.VMEM((BPW,), jnp.int32)]
        + [pltpu.VMEM((C, D_MODEL), jnp.float32)] * NBUF
        + [pltpu.SemaphoreType.DMA] * (2 * NBUF)
    ),
)
def _emb_lookup(idx_hbm, table_hbm, out_hbm, idx_v, *bufs_and_sems):
    bufs = bufs_and_sems[:NBUF]
    gss = bufs_and_sems[NBUF:2 * NBUF]
    sss = bufs_and_sems[2 * NBUF:]

    wid = lax.axis_index("s") * NUM_CORES + lax.axis_index("c")
    base = wid * BPW
    pltpu.sync_copy(idx_hbm.at[pl.ds(base, BPW)], idx_v)

    def start_gather(g, b):
        pltpu.async_copy(table_hbm.at[idx_v.at[pl.ds(g * C, C)]], bufs[b], gss[b])

    def wait_gather(b):
        pltpu.make_async_copy(
            table_hbm.at[idx_v.at[pl.ds(0, C)]], bufs[b], gss[b]
        ).wait()

    def start_scatter(g, b):
        pltpu.async_copy(bufs[b], out_hbm.at[pl.ds(base + g * C, C)], sss[b])

    def wait_scatter(b):
        pltpu.make_async_copy(bufs[b], out_hbm.at[pl.ds(base, C)], sss[b]).wait()

    def scale(b):
        def row_body(r, rc):
            for j in range(D_MODEL // 16):
                sl = pl.ds(j * 16, 16)
                bufs[b][r, sl] = bufs[b][r, sl] * SCALE
            return rc

        lax.fori_loop(0, C, row_body, 0, unroll=False)

    def body(g, b, static):
        wait_gather(b)
        scale(b)
        start_scatter(g, b)
        bn = (b + PF) % NBUF
        if static:
            if g >= NBUF - PF:
                wait_scatter(bn)
            if g < NCHUNK - PF:
                start_gather(g + PF, bn)
        else:
            pl.when(g >= NBUF - PF)(lambda: wait_scatter(bn))
            pl.when(g < NCHUNK - PF)(lambda: start_gather(g + PF, bn))

    for g0 in range(PF):
        start_gather(g0, g0)

    def ring_turn(t, carry):
        for b in range(NBUF):
            body(NBUF * t + b, b, static=False)
        return carry

    lax.fori_loop(0, T, ring_turn, 0, unroll=False)
    for g0 in range(T * NBUF, NCHUNK):
        body(g0, g0 % NBUF, static=True)
    for g0 in range(NCHUNK - (NBUF - PF), NCHUNK):
        wait_scatter(g0 % NBUF)


def kernel(x, table):
    flat_idx = x.reshape(-1)
    out = _emb_lookup(flat_idx, table)
    return out.reshape(x.shape + (D_MODEL,))


# speedup vs baseline: 1.0051x; 1.0051x over previous
import functools
import math

import jax
import jax.numpy as jnp
from jax import lax
from jax.experimental import pallas as pl
from jax.experimental.pallas import tpu as pltpu
from jax.experimental.pallas import tpu_sc as plsc

D_MODEL = 512
SCALE = float(math.sqrt(D_MODEL))

NUM_CORES = 2
NUM_SUBCORES = 16
NW = NUM_CORES * NUM_SUBCORES

B_TOTAL = 1024 * 200
BPW = B_TOTAL // NW
C = 40
NCHUNK = BPW // C
NBUF = 5
PF = 3
T = NCHUNK // NBUF

_mesh = plsc.VectorSubcoreMesh(core_axis_name="c", subcore_axis_name="s")


@functools.partial(
    pl.kernel,
    mesh=_mesh,
    out_type=jax.ShapeDtypeStruct((B_TOTAL, D_MODEL), jnp.float32),
    scratch_types=(
        [pltpu.VMEM((BPW,), jnp.int32)]
        + [pltpu.VMEM((C, D_MODEL), jnp.float32)] * NBUF
        + [pltpu.SemaphoreType.DMA] * (2 * NBUF)
    ),
)
def _emb_lookup(idx_hbm, table_hbm, out_hbm, idx_v, *bufs_and_sems):
    bufs = bufs_and_sems[:NBUF]
    gss = bufs_and_sems[NBUF:2 * NBUF]
    sss = bufs_and_sems[2 * NBUF:]

    wid = lax.axis_index("s") * NUM_CORES + lax.axis_index("c")
    base = wid * BPW
    pltpu.sync_copy(idx_hbm.at[pl.ds(base, BPW)], idx_v)

    def start_gather(g, b):
        pltpu.async_copy(table_hbm.at[idx_v.at[pl.ds(g * C, C)]], bufs[b], gss[b])

    def wait_gather(b):
        pltpu.make_async_copy(
            table_hbm.at[idx_v.at[pl.ds(0, C)]], bufs[b], gss[b]
        ).wait()

    def start_scatter(g, b):
        pltpu.async_copy(bufs[b], out_hbm.at[pl.ds(base + g * C, C)], sss[b])

    def wait_scatter(b):
        pltpu.make_async_copy(bufs[b], out_hbm.at[pl.ds(base, C)], sss[b]).wait()

    def scale(b):
        def row_body(r, rc):
            for j in range(D_MODEL // 16):
                sl = pl.ds(j * 16, 16)
                bufs[b][r, sl] = bufs[b][r, sl] * SCALE
            return rc

        lax.fori_loop(0, C, row_body, 0, unroll=False)

    def body(g, b, static):
        wait_gather(b)
        scale(b)
        start_scatter(g, b)
        bn = (b + PF) % NBUF
        if static:
            if g >= NBUF - PF:
                wait_scatter(bn)
            if g < NCHUNK - PF:
                start_gather(g + PF, bn)
        else:
            pl.when(g >= NBUF - PF)(lambda: wait_scatter(bn))
            pl.when(g < NCHUNK - PF)(lambda: start_gather(g + PF, bn))

    for g0 in range(PF):
        start_gather(g0, g0)

    def ring_turn(t, carry):
        for b in range(NBUF):
            body(NBUF * t + b, b, static=False)
        return carry

    lax.fori_loop(0, T, ring_turn, 0, unroll=False)
    for g0 in range(T * NBUF, NCHUNK):
        body(g0, g0 % NBUF, static=True)
    for g0 in range(NCHUNK - (NBUF - PF), NCHUNK):
        wait_scatter(g0 % NBUF)


def kernel(x, table):
    flat_idx = x.reshape(-1)
    out = _emb_lookup(flat_idx, table)
    return out.reshape(x.shape + (D_MODEL,))


# speedup vs baseline: 1.0093x; 1.0042x over previous
import functools
import math

import jax
import jax.numpy as jnp
from jax import lax
from jax.experimental import pallas as pl
from jax.experimental.pallas import tpu as pltpu
from jax.experimental.pallas import tpu_sc as plsc

D_MODEL = 512
SCALE = float(math.sqrt(D_MODEL))

NUM_CORES = 2
NUM_SUBCORES = 16
NW = NUM_CORES * NUM_SUBCORES

B_TOTAL = 1024 * 200
BPW = B_TOTAL // NW
C = 40
NCHUNK = BPW // C
NBUF = 5
PF = 3
T = NCHUNK // NBUF

_mesh = plsc.VectorSubcoreMesh(core_axis_name="c", subcore_axis_name="s")


@functools.partial(
    pl.kernel,
    mesh=_mesh,
    out_type=jax.ShapeDtypeStruct((B_TOTAL, D_MODEL), jnp.float32),
    scratch_types=(
        [pltpu.VMEM((BPW,), jnp.int32)]
        + [pltpu.VMEM((C, D_MODEL), jnp.float32)] * NBUF
        + [pltpu.SemaphoreType.DMA] * (3 * NBUF)
    ),
)
def _emb_lookup(idx_hbm, table_hbm, out_hbm, idx_v, *bufs_and_sems):
    bufs = bufs_and_sems[:NBUF]
    gss = bufs_and_sems[NBUF:2 * NBUF]
    gss2 = bufs_and_sems[2 * NBUF:3 * NBUF]
    sss = bufs_and_sems[3 * NBUF:]

    wid = lax.axis_index("s") * NUM_CORES + lax.axis_index("c")
    base = wid * BPW
    pltpu.sync_copy(idx_hbm.at[pl.ds(base, BPW)], idx_v)

    CA = 24
    CB = C - CA

    def start_gather(g, b):
        pltpu.async_copy(
            table_hbm.at[idx_v.at[pl.ds(g * C, CA)]],
            bufs[b].at[pl.ds(0, CA)], gss[b])
        pltpu.async_copy(
            table_hbm.at[idx_v.at[pl.ds(g * C + CA, CB)]],
            bufs[b].at[pl.ds(CA, CB)], gss2[b])

    def wait_gather(b):
        pltpu.make_async_copy(
            table_hbm.at[idx_v.at[pl.ds(0, CA)]],
            bufs[b].at[pl.ds(0, CA)], gss[b]
        ).wait()
        pltpu.make_async_copy(
            table_hbm.at[idx_v.at[pl.ds(0, CB)]],
            bufs[b].at[pl.ds(CA, CB)], gss2[b]
        ).wait()

    def start_scatter(g, b):
        pltpu.async_copy(bufs[b], out_hbm.at[pl.ds(base + g * C, C)], sss[b])

    def wait_scatter(b):
        pltpu.make_async_copy(bufs[b], out_hbm.at[pl.ds(base, C)], sss[b]).wait()

    def scale(b):
        def row_body(r, rc):
            for j in range(D_MODEL // 16):
                sl = pl.ds(j * 16, 16)
                bufs[b][r, sl] = bufs[b][r, sl] * SCALE
            return rc

        lax.fori_loop(0, C, row_body, 0, unroll=False)

    def body(g, b, static):
        wait_gather(b)
        scale(b)
        start_scatter(g, b)
        bn = (b + PF) % NBUF
        if static:
            if g >= NBUF - PF:
                wait_scatter(bn)
            if g < NCHUNK - PF:
                start_gather(g + PF, bn)
        else:
            pl.when(g >= NBUF - PF)(lambda: wait_scatter(bn))
            pl.when(g < NCHUNK - PF)(lambda: start_gather(g + PF, bn))

    for g0 in range(PF):
        start_gather(g0, g0)

    def ring_turn(t, carry):
        for b in range(NBUF):
            body(NBUF * t + b, b, static=False)
        return carry

    lax.fori_loop(0, T, ring_turn, 0, unroll=False)
    for g0 in range(T * NBUF, NCHUNK):
        body(g0, g0 % NBUF, static=True)
    for g0 in range(NCHUNK - (NBUF - PF), NCHUNK):
        wait_scatter(g0 % NBUF)


def kernel(x, table):
    flat_idx = x.reshape(-1)
    out = _emb_lookup(flat_idx, table)
    return out.reshape(x.shape + (D_MODEL,))
